# baseline (speedup 1.0000x reference)
import functools
import math

import jax
import jax.numpy as jnp
from jax import lax
from jax.experimental import pallas as pl
from jax.experimental.pallas import tpu as pltpu
from jax.experimental.pallas import tpu_sc as plsc

_LOG_MIN = math.log(0.0 + 1e-08)
_LOG_MAX = math.log(1.0)

_NW = 16
_L = 16


def _make_gather_clip(batch):
    bpw = batch // _NW

    @functools.partial(
        pl.kernel,
        out_type=jax.ShapeDtypeStruct((_NW, bpw), jnp.float32),
        mesh=plsc.VectorSubcoreMesh(
            core_axis_name="c", subcore_axis_name="s", num_cores=1
        ),
        scratch_types=[
            pltpu.VMEM((bpw,), jnp.int32),
            pltpu.VMEM((bpw,), jnp.float32),
            pltpu.SemaphoreType.DMA,
        ],
    )
    def gather_clip(table_hbm, idx_hbm, out_hbm, idx_v, vals_v, sem):
        wid = lax.axis_index("s")
        pltpu.sync_copy(idx_hbm.at[wid], idx_v)
        pltpu.async_copy(table_hbm.at[idx_v], vals_v, sem).wait()
        for k in range(bpw // _L):
            sl = pl.ds(k * _L, _L)
            v = vals_v[sl]
            vals_v[sl] = jnp.minimum(jnp.maximum(v, _LOG_MIN), _LOG_MAX)
        pltpu.sync_copy(vals_v, out_hbm.at[wid])

    return gather_clip


def kernel(log_coefs, coef_idxs):
    batch = coef_idxs.shape[0]
    idx2d = coef_idxs.astype(jnp.int32).reshape(_NW, batch // _NW)
    out = _make_gather_clip(batch)(log_coefs, idx2d)
    return out.reshape(-1, 1)


# speedup vs baseline: 1.0228x; 1.0228x over previous
import functools
import math

import jax
import jax.numpy as jnp
from jax import lax
from jax.experimental import pallas as pl
from jax.experimental.pallas import tpu as pltpu
from jax.experimental.pallas import tpu_sc as plsc

_LOG_MIN = math.log(0.0 + 1e-08)
_LOG_MAX = math.log(1.0)

_NS = 16
_L = 16
_C0 = 704
_C1 = 320


def _make_gather_clip(batch):
    assert (_C0 + _C1) * _NS == batch

    @functools.partial(
        pl.kernel,
        out_type=jax.ShapeDtypeStruct((batch,), jnp.float32),
        mesh=plsc.VectorSubcoreMesh(core_axis_name="c", subcore_axis_name="s"),
        scratch_types=[
            pltpu.VMEM((_C0,), jnp.int32),
            pltpu.VMEM((_C0,), jnp.float32),
            pltpu.VMEM((_C1,), jnp.int32),
            pltpu.VMEM((_C1,), jnp.float32),
            pltpu.SemaphoreType.DMA,
        ],
    )
    def gather_clip(table_hbm, idx_hbm, out_hbm, idx0, vals0, idx1, vals1, sem):
        cid = lax.axis_index("c")
        sid = lax.axis_index("s")

        def run(base, idx_v, vals_v, count):
            pltpu.sync_copy(idx_hbm.at[pl.ds(base, count)], idx_v)
            pltpu.async_copy(table_hbm.at[idx_v], vals_v, sem).wait()
            for k in range(count // _L):
                sl = pl.ds(k * _L, _L)
                v = vals_v[sl]
                vals_v[sl] = jnp.minimum(jnp.maximum(v, _LOG_MIN), _LOG_MAX)
            pltpu.sync_copy(vals_v, out_hbm.at[pl.ds(base, count)])

        @pl.when(cid == 0)
        def _():
            run(sid * _C0, idx0, vals0, _C0)

        @pl.when(cid == 1)
        def _():
            run(_NS * _C0 + sid * _C1, idx1, vals1, _C1)

    return gather_clip


def kernel(log_coefs, coef_idxs):
    batch = coef_idxs.shape[0]
    out = _make_gather_clip(batch)(log_coefs, coef_idxs.astype(jnp.int32))
    return out.reshape(-1, 1)


# speedup vs baseline: 1.0412x; 1.0180x over previous
import functools
import math

import jax
import jax.numpy as jnp
from jax import lax
from jax.experimental import pallas as pl
from jax.experimental.pallas import tpu as pltpu
from jax.experimental.pallas import tpu_sc as plsc

_LOG_MIN = math.log(0.0 + 1e-08)
_LOG_MAX = math.log(1.0)

_NS = 16
_L = 16
_C0 = 320
_C1 = 704


def _make_gather_clip(batch):
    assert (_C0 + _C1) * _NS == batch

    @functools.partial(
        pl.kernel,
        out_type=jax.ShapeDtypeStruct((batch,), jnp.float32),
        mesh=plsc.VectorSubcoreMesh(core_axis_name="c", subcore_axis_name="s"),
        scratch_types=[
            pltpu.VMEM((_C0,), jnp.int32),
            pltpu.VMEM((_C0,), jnp.float32),
            pltpu.VMEM((_C1,), jnp.int32),
            pltpu.VMEM((_C1,), jnp.float32),
            pltpu.SemaphoreType.DMA,
        ],
    )
    def gather_clip(table_hbm, idx_hbm, out_hbm, idx0, vals0, idx1, vals1, sem):
        cid = lax.axis_index("c")
        sid = lax.axis_index("s")

        def run(base, idx_v, vals_v, count):
            pltpu.sync_copy(idx_hbm.at[pl.ds(base, count)], idx_v)
            pltpu.async_copy(table_hbm.at[idx_v], vals_v, sem).wait()
            for k in range(count // _L):
                sl = pl.ds(k * _L, _L)
                v = vals_v[sl]
                vals_v[sl] = jnp.minimum(jnp.maximum(v, _LOG_MIN), _LOG_MAX)
            pltpu.sync_copy(vals_v, out_hbm.at[pl.ds(base, count)])

        @pl.when(cid == 0)
        def _():
            run(sid * _C0, idx0, vals0, _C0)

        @pl.when(cid == 1)
        def _():
            run(_NS * _C0 + sid * _C1, idx1, vals1, _C1)

    return gather_clip


def kernel(log_coefs, coef_idxs):
    batch = coef_idxs.shape[0]
    out = _make_gather_clip(batch)(log_coefs, coef_idxs.astype(jnp.int32))
    return out.reshape(-1, 1)
